# baseline (speedup 1.0000x reference)
import jax
import jax.numpy as jnp
from jax import lax
from jax.experimental import pallas as pl

_B, _T, _D = 16, 2048, 512
_N_CB, _CB_SIZE, _CB_DIM = 9, 1024, 8
_TOK = _B * _T
_HALF = 1024
_BLK = 2 * _HALF
_GRID = _TOK // _BLK
_EPS = 1e-12
_PREC = lax.Precision.DEFAULT


def _rvq_kernel(z_ref, win_ref, inb_ref, wout_ref, outb_ref, cbt2_ref,
                c2_ref, aug_ref,
                zq_ref, codes_ref, lat_ref, loss_ref):
    pid = pl.program_id(0)

    @pl.when(pid == 0)
    def _init_loss():
        loss_ref[...] = jnp.zeros((8, 128), jnp.float32)

    win = win_ref[...]
    inb = inb_ref[0:1, :]
    wout = wout_ref[...]
    cbt2 = cbt2_ref[...]

    z_h = [z_ref[0:_HALF, :], z_ref[_HALF:_BLK, :]]
    res_h = [z_h[0], z_h[1]]
    lats_h = [[], []]
    codes_h = [[], []]
    loss = jnp.zeros((), jnp.float32)

    for i in range(_N_CB):
        sl = slice(8 * i, 8 * (i + 1))
        aug_i = aug_ref[1024 * i:1024 * (i + 1), :]
        for h in range(2):
            z_e = jnp.dot(res_h[h], win[:, sl], precision=_PREC) + inb[:, sl]
            enc_nrm = jnp.sqrt(jnp.sum(z_e * z_e, axis=1, keepdims=True))
            enc_n = z_e / jnp.maximum(_EPS, enc_nrm)
            s2 = jnp.dot(enc_n, cbt2[sl, :], precision=_PREC)
            score = s2 - c2_ref[i:i + 1, :]
            mx = jnp.max(score, axis=1, keepdims=True)
            oh = (score == mx).astype(jnp.bfloat16)
            g = jnp.dot(oh, aug_i, preferred_element_type=jnp.float32)
            z_q_lat = g[:, 0:8]
            idx = (g[:, 8:9] * 256.0 + g[:, 9:10]).astype(jnp.int32)
            diff = z_e - z_q_lat
            loss = loss + jnp.sum(diff * diff)
            z_q_i = jnp.dot(z_q_lat, wout[sl, :], precision=_PREC) \
                + outb_ref[i:i + 1, :]
            res_h[h] = res_h[h] - z_q_i
            lats_h[h].append(z_e)
            codes_h[h].append(idx)

    zq_ref[0:_HALF, :] = z_h[0] - res_h[0]
    zq_ref[_HALF:_BLK, :] = z_h[1] - res_h[1]
    lat_ref[0:_HALF, :] = jnp.concatenate(lats_h[0], axis=1)
    lat_ref[_HALF:_BLK, :] = jnp.concatenate(lats_h[1], axis=1)
    codes_ref[0:_HALF, :] = jnp.concatenate(codes_h[0], axis=1)
    codes_ref[_HALF:_BLK, :] = jnp.concatenate(codes_h[1], axis=1)
    loss_ref[...] += jnp.full((8, 128), loss, jnp.float32)


def kernel(z, in_v, in_g, in_b, out_v, out_g, out_b, codebooks):
    zf = z.reshape(_TOK, _D)
    in_nrm = jnp.sqrt(jnp.sum(in_v * in_v, axis=1, keepdims=True))
    win = in_g[:, None, :] * in_v / in_nrm
    win_cat = win.transpose(1, 0, 2).reshape(_D, _N_CB * _CB_DIM)
    inb = jnp.pad(in_b.reshape(1, -1), ((0, 7), (0, 0)))
    out_nrm = jnp.sqrt(jnp.sum(out_v * out_v, axis=1, keepdims=True))
    wout = out_g[:, None, :] * out_v / out_nrm
    wout_cat = wout.reshape(_N_CB * _CB_DIM, _D)
    outb = jnp.pad(out_b, ((0, 7), (0, 0)))
    cb_nrm = jnp.sqrt(jnp.sum(codebooks * codebooks, axis=2, keepdims=True))
    cb_n = codebooks / jnp.maximum(_EPS, cb_nrm)
    cbt2_cat = (cb_n * 2.0).transpose(0, 2, 1).reshape(_N_CB * _CB_DIM,
                                                       _CB_SIZE)
    c2 = jnp.sum(cb_n * cb_n, axis=2)
    c2_cat = jnp.pad(c2, ((0, 7), (0, 0)))
    ar = jnp.arange(_CB_SIZE, dtype=jnp.float32)
    hi = jnp.floor(ar / 256.0)[:, None]
    lo = (ar - 256.0 * jnp.floor(ar / 256.0))[:, None]
    ones = jnp.ones((_CB_SIZE, 1), jnp.float32)
    zeros = jnp.zeros((_CB_SIZE, 5), jnp.float32)
    aug_one = lambda cb_i: jnp.concatenate(
        [cb_i, hi, lo, ones, zeros], axis=1).astype(jnp.bfloat16)
    aug_cat = jnp.concatenate([aug_one(codebooks[i]) for i in range(_N_CB)],
                              axis=0)

    full = lambda shape: pl.BlockSpec(shape, lambda i: (0,) * len(shape))
    zq_f, codes_f, lat_f, loss_arr = pl.pallas_call(
        _rvq_kernel,
        grid=(_GRID,),
        in_specs=[
            pl.BlockSpec((_BLK, _D), lambda i: (i, 0)),
            full((_D, _N_CB * _CB_DIM)),
            full((8, _N_CB * _CB_DIM)),
            full((_N_CB * _CB_DIM, _D)),
            full((16, _D)),
            full((_N_CB * _CB_DIM, _CB_SIZE)),
            full((16, _CB_SIZE)),
            full((_N_CB * _CB_SIZE, 16)),
        ],
        out_specs=[
            pl.BlockSpec((_BLK, _D), lambda i: (i, 0)),
            pl.BlockSpec((_BLK, _N_CB), lambda i: (i, 0)),
            pl.BlockSpec((_BLK, _N_CB * _CB_DIM), lambda i: (i, 0)),
            full((8, 128)),
        ],
        out_shape=[
            jax.ShapeDtypeStruct((_TOK, _D), jnp.float32),
            jax.ShapeDtypeStruct((_TOK, _N_CB), jnp.int32),
            jax.ShapeDtypeStruct((_TOK, _N_CB * _CB_DIM), jnp.float32),
            jax.ShapeDtypeStruct((8, 128), jnp.float32),
        ],
    )(zf, win_cat, inb, wout_cat, outb, cbt2_cat, c2_cat, aug_cat)

    z_q = zq_f.reshape(_B, _T, _D)
    codes = codes_f.reshape(_B, _T, _N_CB)
    latents = lat_f.reshape(_B, _T, _N_CB * _CB_DIM)
    loss = loss_arr[0, 0] / jnp.float32(_B * _T * _CB_DIM)
    return (z_q, codes, latents, loss, loss)


# speedup vs baseline: 1.0530x; 1.0530x over previous
import jax
import jax.numpy as jnp
from jax import lax
from jax.experimental import pallas as pl

_B, _T, _D = 16, 2048, 512
_N_CB, _CB_SIZE, _CB_DIM = 9, 1024, 8
_TOK = _B * _T
_BLK = 1024
_GRID = _TOK // _BLK
_EPS = 1e-12
_PREC = lax.Precision.DEFAULT


def _rvq_kernel(z_ref, win_ref, inb_ref, wout_ref, outb_ref, cbt2_ref,
                c2_ref, aug_ref,
                zq_ref, codes_ref, lat_ref, loss_ref):
    pid = pl.program_id(0)

    @pl.when(pid == 0)
    def _init_loss():
        loss_ref[...] = jnp.zeros((8, 128), jnp.float32)

    z_in = z_ref[...]
    res = z_in
    win = win_ref[...]
    inb = inb_ref[0:1, :]
    wout = wout_ref[...]
    cbt2 = cbt2_ref[...]

    loss = jnp.zeros((), jnp.float32)
    lats = []
    codes = []

    for i in range(_N_CB):
        sl = slice(8 * i, 8 * (i + 1))
        z_e = jnp.dot(res, win[:, sl], precision=_PREC) + inb[:, sl]
        enc_nrm = jnp.sqrt(jnp.sum(z_e * z_e, axis=1, keepdims=True))
        enc_n = z_e / jnp.maximum(_EPS, enc_nrm)
        s2 = jnp.dot(enc_n, cbt2[sl, :], precision=_PREC)
        score = s2 - c2_ref[i:i + 1, :]
        mx = jnp.max(score, axis=1, keepdims=True)
        oh = (score == mx).astype(jnp.bfloat16)
        aug_i = aug_ref[1024 * i:1024 * (i + 1), :]
        g = jnp.dot(oh, aug_i, preferred_element_type=jnp.float32)
        z_q_lat = g[:, 0:8]
        idx = g[:, 8:10]
        diff = z_e - z_q_lat
        loss = loss + jnp.sum(diff * diff)
        z_q_i = jnp.dot(z_q_lat, wout[sl, :], precision=_PREC) \
            + outb_ref[i:i + 1, :]
        res = res - z_q_i
        lats.append(z_e)
        codes.append(idx)

    zq_ref[...] = z_in - res
    lat_ref[...] = jnp.concatenate(lats, axis=1)
    codes_ref[...] = jnp.concatenate(codes, axis=1)
    loss_ref[...] += jnp.full((8, 128), loss, jnp.float32)


def kernel(z, in_v, in_g, in_b, out_v, out_g, out_b, codebooks):
    zf = z.reshape(_TOK, _D)
    in_nrm = jnp.sqrt(jnp.sum(in_v * in_v, axis=1, keepdims=True))
    win = in_g[:, None, :] * in_v / in_nrm
    win_cat = win.transpose(1, 0, 2).reshape(_D, _N_CB * _CB_DIM)
    inb = jnp.pad(in_b.reshape(1, -1), ((0, 7), (0, 0)))
    out_nrm = jnp.sqrt(jnp.sum(out_v * out_v, axis=1, keepdims=True))
    wout = out_g[:, None, :] * out_v / out_nrm
    wout_cat = wout.reshape(_N_CB * _CB_DIM, _D)
    outb = jnp.pad(out_b, ((0, 7), (0, 0)))
    cb_nrm = jnp.sqrt(jnp.sum(codebooks * codebooks, axis=2, keepdims=True))
    cb_n = codebooks / jnp.maximum(_EPS, cb_nrm)
    cbt2_cat = (cb_n * 2.0).transpose(0, 2, 1).reshape(_N_CB * _CB_DIM,
                                                       _CB_SIZE)
    c2 = jnp.sum(cb_n * cb_n, axis=2)
    c2_cat = jnp.pad(c2, ((0, 7), (0, 0)))
    ar = jnp.arange(_CB_SIZE, dtype=jnp.float32)
    hi = jnp.floor(ar / 256.0)[:, None]
    lo = (ar - 256.0 * jnp.floor(ar / 256.0))[:, None]
    ones = jnp.ones((_CB_SIZE, 1), jnp.float32)
    zeros = jnp.zeros((_CB_SIZE, 5), jnp.float32)
    aug_one = lambda cb_i: jnp.concatenate(
        [cb_i, hi, lo, ones, zeros], axis=1).astype(jnp.bfloat16)
    aug_cat = jnp.concatenate([aug_one(codebooks[i]) for i in range(_N_CB)],
                              axis=0)

    full = lambda shape: pl.BlockSpec(shape, lambda i: (0,) * len(shape))
    zq_f, codes_f, lat_f, loss_arr = pl.pallas_call(
        _rvq_kernel,
        grid=(_GRID,),
        in_specs=[
            pl.BlockSpec((_BLK, _D), lambda i: (i, 0)),
            full((_D, _N_CB * _CB_DIM)),
            full((8, _N_CB * _CB_DIM)),
            full((_N_CB * _CB_DIM, _D)),
            full((16, _D)),
            full((_N_CB * _CB_DIM, _CB_SIZE)),
            full((16, _CB_SIZE)),
            full((_N_CB * _CB_SIZE, 16)),
        ],
        out_specs=[
            pl.BlockSpec((_BLK, _D), lambda i: (i, 0)),
            pl.BlockSpec((_BLK, 2 * _N_CB), lambda i: (i, 0)),
            pl.BlockSpec((_BLK, _N_CB * _CB_DIM), lambda i: (i, 0)),
            full((8, 128)),
        ],
        out_shape=[
            jax.ShapeDtypeStruct((_TOK, _D), jnp.float32),
            jax.ShapeDtypeStruct((_TOK, 2 * _N_CB), jnp.float32),
            jax.ShapeDtypeStruct((_TOK, _N_CB * _CB_DIM), jnp.float32),
            jax.ShapeDtypeStruct((8, 128), jnp.float32),
        ],
    )(zf, win_cat, inb, wout_cat, outb, cbt2_cat, c2_cat, aug_cat)

    z_q = zq_f.reshape(_B, _T, _D)
    codes = (codes_f[:, 0::2] * 256.0
             + codes_f[:, 1::2]).astype(jnp.int32).reshape(_B, _T, _N_CB)
    latents = lat_f.reshape(_B, _T, _N_CB * _CB_DIM)
    loss = loss_arr[0, 0] / jnp.float32(_B * _T * _CB_DIM)
    return (z_q, codes, latents, loss, loss)


# speedup vs baseline: 1.3031x; 1.2376x over previous
import jax
import jax.numpy as jnp
from jax import lax
from jax.experimental import pallas as pl

_B, _T, _D = 16, 2048, 512
_N_CB, _CB_SIZE, _CB_DIM = 9, 1024, 8
_TOK = _B * _T
_BLK = 1024
_GRID = _TOK // _BLK
_EPS = 1e-12
_PREC = lax.Precision.DEFAULT


def _rvq_kernel(z_ref, win_ref, inb_ref, wout_ref, outb_ref, cbt2_ref,
                c2_ref, aug_ref,
                zq_ref, codes_ref, lat_ref, loss_ref):
    pid = pl.program_id(0)

    @pl.when(pid == 0)
    def _init_loss():
        loss_ref[...] = jnp.zeros((8, 128), jnp.float32)

    z_in = z_ref[...]
    res = z_in
    win = win_ref[...]
    inb = inb_ref[0:1, :]
    wout = wout_ref[...]
    cbt2 = cbt2_ref[...]

    loss = jnp.zeros((), jnp.float32)
    lats = []
    codes = []

    for i in range(_N_CB):
        sl = slice(8 * i, 8 * (i + 1))
        z_e = jnp.dot(res, win[:, sl], precision=_PREC) + inb[:, sl]
        enc_nrm = jnp.sqrt(jnp.sum(z_e * z_e, axis=1, keepdims=True))
        enc_n = z_e / jnp.maximum(_EPS, enc_nrm)
        s2 = jnp.dot(enc_n, cbt2[sl, :], precision=_PREC)
        score = s2 - c2_ref[i:i + 1, :]
        mx = jnp.max(score, axis=1, keepdims=True)
        oh = (score == mx).astype(jnp.bfloat16)
        aug_i = aug_ref[1024 * i:1024 * (i + 1), :]
        g = jnp.dot(oh, aug_i, preferred_element_type=jnp.float32)
        z_q_lat = g[:, 0:8]
        idx = (g[:, 8:9] * 256.0 + g[:, 9:10]).astype(jnp.int32)
        diff = z_e - z_q_lat
        loss = loss + jnp.sum(diff * diff)
        z_q_i = jnp.dot(z_q_lat, wout[sl, :], precision=_PREC) \
            + outb_ref[i:i + 1, :]
        res = res - z_q_i
        lats.append(z_e)
        codes.append(idx)

    zq_ref[...] = z_in - res
    lat_ref[...] = jnp.concatenate(lats, axis=1)
    codes_ref[...] = jnp.concatenate(codes, axis=1)
    loss_ref[...] += jnp.full((8, 128), loss, jnp.float32)


def kernel(z, in_v, in_g, in_b, out_v, out_g, out_b, codebooks):
    zf = z.reshape(_TOK, _D)
    in_nrm = jnp.sqrt(jnp.sum(in_v * in_v, axis=1, keepdims=True))
    win = in_g[:, None, :] * in_v / in_nrm
    win_cat = win.transpose(1, 0, 2).reshape(_D, _N_CB * _CB_DIM)
    inb = jnp.pad(in_b.reshape(1, -1), ((0, 7), (0, 0)))
    out_nrm = jnp.sqrt(jnp.sum(out_v * out_v, axis=1, keepdims=True))
    wout = out_g[:, None, :] * out_v / out_nrm
    wout_cat = wout.reshape(_N_CB * _CB_DIM, _D)
    outb = jnp.pad(out_b, ((0, 7), (0, 0)))
    cb_nrm = jnp.sqrt(jnp.sum(codebooks * codebooks, axis=2, keepdims=True))
    cb_n = codebooks / jnp.maximum(_EPS, cb_nrm)
    cbt2_cat = (cb_n * 2.0).transpose(0, 2, 1).reshape(_N_CB * _CB_DIM,
                                                       _CB_SIZE)
    c2 = jnp.sum(cb_n * cb_n, axis=2)
    c2_cat = jnp.pad(c2, ((0, 7), (0, 0)))
    ar = jnp.arange(_CB_SIZE, dtype=jnp.float32)
    hi = jnp.floor(ar / 256.0)[:, None]
    lo = (ar - 256.0 * jnp.floor(ar / 256.0))[:, None]
    ones = jnp.ones((_CB_SIZE, 1), jnp.float32)
    zeros = jnp.zeros((_CB_SIZE, 5), jnp.float32)
    aug_one = lambda cb_i: jnp.concatenate(
        [cb_i, hi, lo, ones, zeros], axis=1).astype(jnp.bfloat16)
    aug_cat = jnp.concatenate([aug_one(codebooks[i]) for i in range(_N_CB)],
                              axis=0)

    full = lambda shape: pl.BlockSpec(shape, lambda i: (0,) * len(shape))
    zq_f, codes_f, lat_f, loss_arr = pl.pallas_call(
        _rvq_kernel,
        grid=(_GRID,),
        in_specs=[
            pl.BlockSpec((_BLK, _D), lambda i: (i, 0)),
            full((_D, _N_CB * _CB_DIM)),
            full((8, _N_CB * _CB_DIM)),
            full((_N_CB * _CB_DIM, _D)),
            full((16, _D)),
            full((_N_CB * _CB_DIM, _CB_SIZE)),
            full((16, _CB_SIZE)),
            full((_N_CB * _CB_SIZE, 16)),
        ],
        out_specs=[
            pl.BlockSpec((_BLK, _D), lambda i: (i, 0)),
            pl.BlockSpec((_BLK, _N_CB), lambda i: (i, 0)),
            pl.BlockSpec((_BLK, _N_CB * _CB_DIM), lambda i: (i, 0)),
            full((8, 128)),
        ],
        out_shape=[
            jax.ShapeDtypeStruct((_TOK, _D), jnp.float32),
            jax.ShapeDtypeStruct((_TOK, _N_CB), jnp.int32),
            jax.ShapeDtypeStruct((_TOK, _N_CB * _CB_DIM), jnp.float32),
            jax.ShapeDtypeStruct((8, 128), jnp.float32),
        ],
    )(zf, win_cat, inb, wout_cat, outb, cbt2_cat, c2_cat, aug_cat)

    z_q = zq_f.reshape(_B, _T, _D)
    codes = codes_f.reshape(_B, _T, _N_CB)
    latents = lat_f.reshape(_B, _T, _N_CB * _CB_DIM)
    loss = loss_arr[0, 0] / jnp.float32(_B * _T * _CB_DIM)
    return (z_q, codes, latents, loss, loss)
